# baseline (speedup 1.0000x reference)
import functools

import jax
import jax.numpy as jnp
from jax import lax
from jax.experimental import pallas as pl
from jax.experimental.pallas import tpu as pltpu
from jax.experimental.pallas import tpu_sc as plsc

M = 256
N = 256
DIM = 64
R = M * N
DECAY = 0.999
ALPHA = 0.3
SIGMA = max(M, N) / 2.0

NC = 2
NS = 16
NW = NC * NS
ROWS_W = R // NW
R2 = R // 2
ROWS2_W = R2 // NW
CH = 128
NCHUNK = ROWS2_W // CH
GRP = 2 * CH // 16

_mesh = plsc.VectorSubcoreMesh(
    core_axis_name="c", subcore_axis_name="s", num_cores=NC, num_subcores=NS
)

_DNUMS = lax.GatherDimensionNumbers(
    offset_dims=(), collapsed_slice_dims=(0,), start_index_map=(0,)
)


def _perm(v, idx):
    return lax.gather(
        v, idx[:, None], _DNUMS, (1,), mode=lax.GatherScatterMode.PROMISE_IN_BOUNDS
    )


def _tree(v, op):
    i = lax.iota(jnp.int32, 16)
    for sh in (8, 4, 2, 1):
        v = op(v, _perm(v, i ^ sh))
    return v


def _worker_base():
    wid = lax.axis_index("s") * NC + lax.axis_index("c")
    return wid, wid * ROWS2_W


@functools.partial(
    pl.kernel,
    out_type=[
        jax.ShapeDtypeStruct((NW, 16), jnp.float32),
        jax.ShapeDtypeStruct((NW, 16), jnp.int32),
    ],
    mesh=_mesh,
    scratch_types=[
        pltpu.VMEM((DIM,), jnp.float32),
        pltpu.VMEM((CH, 128), jnp.float32),
        pltpu.VMEM((CH, 128), jnp.float32),
        pltpu.VMEM((1, 16), jnp.float32),
        pltpu.VMEM((1, 16), jnp.int32),
        pltpu.SemaphoreType.DMA,
        pltpu.SemaphoreType.DMA,
    ],
)
def _bmu_kernel(x_hbm, w_hbm, mind_hbm, mini_hbm, xv, b0, b1, outd, outi, s0, s1):
    wid, base = _worker_base()
    pltpu.sync_copy(x_hbm, xv)
    x2 = [2.0 * xv[pl.ds(16 * q, 16)] for q in range(4)]

    bufs = [b0, b1]
    sems = [s0, s1]
    cps = [None, None]
    cps[0] = pltpu.async_copy(w_hbm.at[pl.ds(base, CH)], bufs[0], sems[0])

    md = jnp.full((16,), jnp.inf, jnp.float32)
    mi = jnp.zeros((16,), jnp.int32)
    for c in range(NCHUNK):
        cps[c % 2].wait()
        if c + 1 < NCHUNK:
            cps[(c + 1) % 2] = pltpu.async_copy(
                w_hbm.at[pl.ds(base + (c + 1) * CH, CH)],
                bufs[(c + 1) % 2],
                sems[(c + 1) % 2],
            )
        cur = bufs[c % 2]

        def grp(g, carry, cur=cur, c=c):
            md, mi = carry
            for j in range(16):
                r128 = g * 8 + (j >> 1)
                half = (j & 1) * 64
                p = None
                for q in range(4):
                    wq = cur[r128, pl.ds(half + 16 * q, 16)]
                    t = wq * (wq - x2[q])
                    p = t if p is None else p + t
                s = _tree(p, jnp.add)
                bet = s < md
                md = jnp.where(bet, s, md)
                rglob = (base + c * CH) * 2 + g * 16 + j
                mi = jnp.where(bet, jnp.full((16,), rglob, jnp.int32), mi)
            return md, mi

        md, mi = lax.fori_loop(0, GRP, grp, (md, mi))

    outd[0] = md
    outi[0] = mi
    pltpu.sync_copy(outd, mind_hbm.at[pl.ds(wid, 1)])
    pltpu.sync_copy(outi, mini_hbm.at[pl.ds(wid, 1)])


@functools.partial(
    pl.kernel,
    out_type=jax.ShapeDtypeStruct((R2, 128), jnp.float32),
    mesh=_mesh,
    scratch_types=[
        pltpu.VMEM((DIM,), jnp.float32),
        pltpu.VMEM((2, 16), jnp.float32),
        pltpu.VMEM((NW, 16), jnp.float32),
        pltpu.VMEM((NW, 16), jnp.int32),
        pltpu.VMEM((CH, 128), jnp.float32),
        pltpu.VMEM((CH, 128), jnp.float32),
        pltpu.VMEM((CH, 128), jnp.float32),
        pltpu.VMEM((CH, 128), jnp.float32),
        pltpu.SemaphoreType.DMA,
        pltpu.SemaphoreType.DMA,
        pltpu.SemaphoreType.DMA,
        pltpu.SemaphoreType.DMA,
    ],
)
def _update_kernel(
    x_hbm, w_hbm, p_hbm, mind_hbm, mini_hbm, out_hbm,
    xv, pv, mdv, miv, i0, i1, o0, o1, si0, si1, so0, so1,
):
    wid, base = _worker_base()
    pltpu.sync_copy(x_hbm, xv)
    pltpu.sync_copy(p_hbm, pv)
    pltpu.sync_copy(mind_hbm, mdv)
    pltpu.sync_copy(mini_hbm, miv)

    xs = [xv[pl.ds(16 * q, 16)] for q in range(4)]
    av = pv[0]
    cv = pv[1]

    bd = mdv[0]
    bi = miv[0]
    for j in range(1, NW):
        dv = mdv[j]
        iv = miv[j]
        bet = (dv < bd) | ((dv == bd) & (iv < bi))
        bd = jnp.where(bet, dv, bd)
        bi = jnp.where(bet, iv, bi)
    m = _tree(bd, jnp.minimum)
    cand = jnp.where(bd == m, bi, jnp.int32(1 << 30))
    bmu = _tree(cand, jnp.minimum)
    bxv = bmu & (M - 1)
    byv = bmu >> 8

    ibufs = [i0, i1]
    obufs = [o0, o1]
    isems = [si0, si1]
    osems = [so0, so1]
    icp = [None, None]
    ocp = [None, None]
    icp[0] = pltpu.async_copy(w_hbm.at[pl.ds(base, CH)], ibufs[0], isems[0])

    for c in range(NCHUNK):
        icp[c % 2].wait()
        if c + 1 < NCHUNK:
            icp[(c + 1) % 2] = pltpu.async_copy(
                w_hbm.at[pl.ds(base + (c + 1) * CH, CH)],
                ibufs[(c + 1) % 2],
                isems[(c + 1) % 2],
            )
        if c >= 2:
            ocp[c % 2].wait()
        cur = ibufs[c % 2]
        ob = obufs[c % 2]

        def grp(g, carry, cur=cur, ob=ob, c=c):
            riota = (base + c * CH) * 2 + g * 16 + lax.iota(jnp.int32, 16)
            dx = (riota & (M - 1)) - bxv
            dy = (riota >> 8) - byv
            d2f = (dx * dx + dy * dy).astype(jnp.float32)
            lrv = av * jnp.exp(d2f * cv)
            for j in range(16):
                r128 = g * 8 + (j >> 1)
                half = (j & 1) * 64
                ls = _perm(lrv, jnp.full((16,), j, jnp.int32))
                for q in range(4):
                    wq = cur[r128, pl.ds(half + 16 * q, 16)]
                    ob[r128, pl.ds(half + 16 * q, 16)] = wq + ls * (xs[q] - wq)
            return carry

        lax.fori_loop(0, GRP, grp, 0)
        ocp[c % 2] = pltpu.async_copy(
            ob, out_hbm.at[pl.ds(base + c * CH, CH)], osems[c % 2]
        )

    ocp[0].wait()
    ocp[1].wait()


def kernel(x, step, weights, loc_x, loc_y):
    decay = DECAY ** step
    alpha_op = ALPHA * decay
    sigma_op = SIGMA * decay
    coef = -1.0 / (2.0 * sigma_op * sigma_op)
    params = jnp.stack(
        [
            jnp.full((16,), alpha_op, jnp.float32),
            jnp.full((16,), coef, jnp.float32),
        ]
    )
    w2 = weights.reshape(R2, 128)
    mind, mini = _bmu_kernel(x, w2)
    out = _update_kernel(x, w2, params, mind, mini)
    return out.reshape(R, DIM)


# speedup vs baseline: 1.2283x; 1.2283x over previous
import functools

import jax
import jax.numpy as jnp
from jax import lax
from jax.experimental import pallas as pl
from jax.experimental.pallas import tpu as pltpu
from jax.experimental.pallas import tpu_sc as plsc

M = 256
N = 256
DIM = 64
R = M * N
DECAY = 0.999
ALPHA = 0.3
SIGMA = max(M, N) / 2.0

NC = 2
NS = 16
NW = NC * NS
ROWS_W = R // NW
CH = 256
NCHUNK = ROWS_W // CH
GRP = CH // 16

_mesh = plsc.VectorSubcoreMesh(
    core_axis_name="c", subcore_axis_name="s", num_cores=NC, num_subcores=NS
)

_DNUMS = lax.GatherDimensionNumbers(
    offset_dims=(), collapsed_slice_dims=(0,), start_index_map=(0,)
)


def _perm(v, idx):
    return lax.gather(
        v, idx[:, None], _DNUMS, (1,), mode=lax.GatherScatterMode.PROMISE_IN_BOUNDS
    )


def _tree(v, op):
    i = lax.iota(jnp.int32, 16)
    for sh in (8, 4, 2, 1):
        v = op(v, _perm(v, i ^ sh))
    return v


def _worker_base():
    wid = lax.axis_index("s") * NC + lax.axis_index("c")
    return wid, wid * ROWS_W


@functools.partial(
    pl.kernel,
    out_type=[
        jax.ShapeDtypeStruct((NW, 16), jnp.float32),
        jax.ShapeDtypeStruct((NW, 16), jnp.int32),
    ],
    mesh=_mesh,
    scratch_types=[
        pltpu.VMEM((DIM,), jnp.float32),
        pltpu.VMEM((CH, DIM), jnp.float32),
        pltpu.VMEM((CH, DIM), jnp.float32),
        pltpu.VMEM((1, 16), jnp.float32),
        pltpu.VMEM((1, 16), jnp.int32),
        pltpu.SemaphoreType.DMA,
        pltpu.SemaphoreType.DMA,
    ],
)
def _bmu_kernel(x_hbm, w_hbm, mind_hbm, mini_hbm, xv, b0, b1, outd, outi, s0, s1):
    wid, base = _worker_base()
    pltpu.sync_copy(x_hbm, xv)
    x2 = [2.0 * xv[pl.ds(16 * q, 16)] for q in range(4)]

    bufs = [b0, b1]
    sems = [s0, s1]
    cps = [None, None]
    cps[0] = pltpu.async_copy(w_hbm.at[pl.ds(base, CH)], bufs[0], sems[0])

    md = jnp.full((16,), jnp.inf, jnp.float32)
    mi = jnp.zeros((16,), jnp.int32)
    for c in range(NCHUNK):
        cps[c % 2].wait()
        if c + 1 < NCHUNK:
            cps[(c + 1) % 2] = pltpu.async_copy(
                w_hbm.at[pl.ds(base + (c + 1) * CH, CH)],
                bufs[(c + 1) % 2],
                sems[(c + 1) % 2],
            )
        cur = bufs[c % 2]

        def grp(g, carry, cur=cur, c=c):
            md, mi = carry
            for j in range(16):
                r = g * 16 + j
                p = None
                for q in range(4):
                    wq = cur[r, pl.ds(16 * q, 16)]
                    t = wq * (wq - x2[q])
                    p = t if p is None else p + t
                s = _tree(p, jnp.add)
                bet = s < md
                md = jnp.where(bet, s, md)
                rglob = base + c * CH + r
                mi = jnp.where(bet, jnp.full((16,), rglob, jnp.int32), mi)
            return md, mi

        md, mi = lax.fori_loop(0, GRP, grp, (md, mi))

    outd[0] = md
    outi[0] = mi
    pltpu.sync_copy(outd, mind_hbm.at[pl.ds(wid, 1)])
    pltpu.sync_copy(outi, mini_hbm.at[pl.ds(wid, 1)])


@functools.partial(
    pl.kernel,
    out_type=jax.ShapeDtypeStruct((R, DIM), jnp.float32),
    mesh=_mesh,
    scratch_types=[
        pltpu.VMEM((DIM,), jnp.float32),
        pltpu.VMEM((2, 16), jnp.float32),
        pltpu.VMEM((NW, 16), jnp.float32),
        pltpu.VMEM((NW, 16), jnp.int32),
        pltpu.VMEM((CH, DIM), jnp.float32),
        pltpu.VMEM((CH, DIM), jnp.float32),
        pltpu.VMEM((CH, DIM), jnp.float32),
        pltpu.SemaphoreType.DMA,
        pltpu.SemaphoreType.DMA,
        pltpu.SemaphoreType.DMA,
        pltpu.SemaphoreType.DMA,
        pltpu.SemaphoreType.DMA,
        pltpu.SemaphoreType.DMA,
    ],
)
def _update_kernel(
    x_hbm, w_hbm, p_hbm, mind_hbm, mini_hbm, out_hbm,
    xv, pv, mdv, miv, b0, b1, b2, si0, si1, si2, so0, so1, so2,
):
    wid, base = _worker_base()
    pltpu.sync_copy(x_hbm, xv)
    pltpu.sync_copy(p_hbm, pv)
    pltpu.sync_copy(mind_hbm, mdv)
    pltpu.sync_copy(mini_hbm, miv)

    xs = [xv[pl.ds(16 * q, 16)] for q in range(4)]
    av = pv[0]
    cv = pv[1]

    bd = mdv[0]
    bi = miv[0]
    for j in range(1, NW):
        dv = mdv[j]
        iv = miv[j]
        bet = (dv < bd) | ((dv == bd) & (iv < bi))
        bd = jnp.where(bet, dv, bd)
        bi = jnp.where(bet, iv, bi)
    m = _tree(bd, jnp.minimum)
    cand = jnp.where(bd == m, bi, jnp.int32(1 << 30))
    bmu = _tree(cand, jnp.minimum)
    bxv = bmu & (M - 1)
    byv = bmu >> 8

    bufs = [b0, b1, b2]
    isems = [si0, si1, si2]
    osems = [so0, so1, so2]
    icp = [None] * NCHUNK
    ocp = [None] * NCHUNK
    icp[0] = pltpu.async_copy(w_hbm.at[pl.ds(base, CH)], bufs[0], isems[0])
    icp[1] = pltpu.async_copy(w_hbm.at[pl.ds(base + CH, CH)], bufs[1], isems[1])

    for c in range(NCHUNK):
        icp[c].wait()
        buf = bufs[c % 3]

        def grp(g, carry, buf=buf, c=c):
            riota = base + c * CH + g * 16 + lax.iota(jnp.int32, 16)
            dx = (riota & (M - 1)) - bxv
            dy = (riota >> 8) - byv
            d2f = (dx * dx + dy * dy).astype(jnp.float32)
            lrv = av * jnp.exp(d2f * cv)
            for j in range(16):
                r = g * 16 + j
                ls = _perm(lrv, jnp.full((16,), j, jnp.int32))
                for q in range(4):
                    wq = buf[r, pl.ds(16 * q, 16)]
                    buf[r, pl.ds(16 * q, 16)] = wq + ls * (xs[q] - wq)
            return carry

        lax.fori_loop(0, GRP, grp, 0)
        ocp[c] = pltpu.async_copy(
            buf, out_hbm.at[pl.ds(base + c * CH, CH)], osems[c % 3]
        )
        if c + 2 < NCHUNK:
            if c >= 1:
                ocp[c - 1].wait()
            icp[c + 2] = pltpu.async_copy(
                w_hbm.at[pl.ds(base + (c + 2) * CH, CH)],
                bufs[(c + 2) % 3],
                isems[(c + 2) % 3],
            )

    ocp[NCHUNK - 3].wait()
    ocp[NCHUNK - 2].wait()
    ocp[NCHUNK - 1].wait()


def kernel(x, step, weights, loc_x, loc_y):
    decay = DECAY ** step
    alpha_op = ALPHA * decay
    sigma_op = SIGMA * decay
    coef = -1.0 / (2.0 * sigma_op * sigma_op)
    params = jnp.stack(
        [
            jnp.full((16,), alpha_op, jnp.float32),
            jnp.full((16,), coef, jnp.float32),
        ]
    )
    mind, mini = _bmu_kernel(x, weights)
    return _update_kernel(x, weights, params, mind, mini)
